# baseline (speedup 1.0000x reference)
import functools

import jax
import jax.numpy as jnp
from jax import lax
from jax.experimental import pallas as pl
from jax.experimental.pallas import tpu as pltpu
from jax.experimental.pallas import tpu_sc as plsc

_NUM_CODES = 8192
_DIM = 32
_TB = 256
_CH = 128


def _vq_argmin_body(x_ref, cb_ref, idx_ref, cbt_ref):
    cb = cb_ref[...]
    b = jnp.sum(cb * cb, axis=0, keepdims=True)
    cbm2 = -2.0 * cb
    cbt_ref[...] = cb.T

    nchunks = _NUM_CODES // _CH
    lane = jax.lax.broadcasted_iota(jnp.int32, (_TB, _CH), 1)

    def block(t, carry):
        x = x_ref[pl.ds(t * _TB, _TB), :]
        a = jnp.sum(x * x, axis=1, keepdims=True)
        m2 = jnp.dot(x, cbm2, preferred_element_type=jnp.float32)

        run_min = jnp.full((_TB, _CH), jnp.inf, dtype=jnp.float32)
        run_chunk = jnp.zeros((_TB, _CH), dtype=jnp.int32)
        for j in range(nchunks):
            sl = slice(j * _CH, (j + 1) * _CH)
            nrm = (a + b[:, sl]) + m2[:, sl]
            lt = nrm < run_min
            run_min = jnp.where(lt, nrm, run_min)
            run_chunk = jnp.where(lt, j, run_chunk)

        gmin = jnp.min(run_min, axis=1, keepdims=True)
        cand = run_chunk * _CH + lane
        idx = jnp.min(jnp.where(run_min == gmin, cand, _NUM_CODES), axis=1)
        idx_ref[pl.ds(t, 1), :, :] = idx.reshape(1, 1, _TB)
        return carry

    lax.fori_loop(0, x_ref.shape[0] // _TB, block, 0)


def _tc_argmin(x, codebook):
    n = x.shape[0]
    idx, cbt = pl.pallas_call(
        _vq_argmin_body,
        out_shape=(
            jax.ShapeDtypeStruct((n // _TB, 1, _TB), jnp.int32),
            jax.ShapeDtypeStruct((_NUM_CODES, _DIM), jnp.float32),
        ),
    )(x, codebook)
    return idx.reshape(n), cbt


def _sc_gather(table, idx):
    B = idx.shape[0]
    info = plsc.get_sparse_core_info()
    nw = info.num_cores * info.num_subcores
    b_per_w = B // nw
    nseg = b_per_w // _CH
    idx3 = idx.reshape(nw, nseg, _CH)
    mesh = plsc.VectorSubcoreMesh(core_axis_name="c", subcore_axis_name="s")

    @functools.partial(
        pl.kernel, mesh=mesh,
        compiler_params=pltpu.CompilerParams(use_tc_tiling_on_sc=False),
        out_type=jax.ShapeDtypeStruct((B, _DIM), jnp.float32),
        scratch_types=[
            pltpu.VMEM((nseg, _CH), jnp.int32),
            pltpu.VMEM((b_per_w, _DIM), jnp.float32),
            pltpu.SemaphoreType.DMA,
        ],
    )
    def k(table_hbm, idx_hbm, out_hbm, idx_v, rows_v, sem):
        wid = lax.axis_index("s") * info.num_cores + lax.axis_index("c")
        pltpu.sync_copy(idx_hbm.at[wid], idx_v)
        copies = [
            pltpu.async_copy(
                table_hbm.at[idx_v.at[s]],
                rows_v.at[pl.ds(s * _CH, _CH)],
                sem,
            )
            for s in range(nseg)
        ]
        for c in copies:
            c.wait()
        pltpu.sync_copy(rows_v, out_hbm.at[pl.ds(wid * b_per_w, b_per_w)])

    return k(table, idx3)


def kernel(inputs, codebook):
    original_shape = inputs.shape
    x = inputs.reshape(-1, _DIM)
    idx, cbt = _tc_argmin(x, codebook)
    q = _sc_gather(cbt, idx)
    return q.reshape(original_shape)


# speedup vs baseline: 1.0160x; 1.0160x over previous
import functools

import jax
import jax.numpy as jnp
from jax import lax
from jax.experimental import pallas as pl
from jax.experimental.pallas import tpu as pltpu
from jax.experimental.pallas import tpu_sc as plsc

_NUM_CODES = 8192
_DIM = 32
_TB = 512
_CH = 128


def _vq_argmin_block(x_ref, cb_ref, idx_ref, cbt_ref, b_ref, cbm2_ref):
    @pl.when(pl.program_id(0) == 0)
    def _():
        cb0 = cb_ref[...]
        b_ref[...] = jnp.sum(cb0 * cb0, axis=0, keepdims=True)
        cbm2_ref[...] = -2.0 * cb0
        cbt_ref[...] = cb0.T

    x = x_ref[...]
    a = jnp.sum(x * x, axis=1, keepdims=True)
    m2 = jnp.dot(x, cbm2_ref[...], preferred_element_type=jnp.float32)
    b = b_ref[...]

    nchunks = _NUM_CODES // _CH
    run_min = jnp.full((_TB, _CH), jnp.inf, dtype=jnp.float32)
    run_chunk = jnp.zeros((_TB, _CH), dtype=jnp.int32)
    for j in range(nchunks):
        sl = slice(j * _CH, (j + 1) * _CH)
        nrm = (a + b[:, sl]) + m2[:, sl]
        lt = nrm < run_min
        run_min = jnp.where(lt, nrm, run_min)
        run_chunk = jnp.where(lt, j, run_chunk)

    gmin = jnp.min(run_min, axis=1, keepdims=True)
    lane = jax.lax.broadcasted_iota(jnp.int32, (_TB, _CH), 1)
    cand = run_chunk * _CH + lane
    idx = jnp.min(jnp.where(run_min == gmin, cand, _NUM_CODES), axis=1)
    idx_ref[...] = idx.reshape(1, 1, _TB)


def _tc_argmin(x, codebook):
    n = x.shape[0]
    grid = (n // _TB,)
    idx, cbt = pl.pallas_call(
        _vq_argmin_block,
        grid=grid,
        in_specs=[
            pl.BlockSpec((_TB, _DIM), lambda i: (i, 0)),
            pl.BlockSpec((_DIM, _NUM_CODES), lambda i: (0, 0)),
        ],
        out_specs=[
            pl.BlockSpec((1, 1, _TB), lambda i: (i, 0, 0)),
            pl.BlockSpec((_NUM_CODES, _DIM), lambda i: (0, 0)),
        ],
        out_shape=(
            jax.ShapeDtypeStruct((n // _TB, 1, _TB), jnp.int32),
            jax.ShapeDtypeStruct((_NUM_CODES, _DIM), jnp.float32),
        ),
        scratch_shapes=[
            pltpu.VMEM((1, _NUM_CODES), jnp.float32),
            pltpu.VMEM((_DIM, _NUM_CODES), jnp.float32),
        ],
    )(x, codebook)
    return idx.reshape(n), cbt


def _sc_gather(table, idx):
    B = idx.shape[0]
    info = plsc.get_sparse_core_info()
    nw = info.num_cores * info.num_subcores
    b_per_w = B // nw
    nseg = b_per_w // _CH
    idx3 = idx.reshape(nw, nseg, _CH)
    mesh = plsc.VectorSubcoreMesh(core_axis_name="c", subcore_axis_name="s")

    @functools.partial(
        pl.kernel, mesh=mesh,
        compiler_params=pltpu.CompilerParams(use_tc_tiling_on_sc=False),
        out_type=jax.ShapeDtypeStruct((B, _DIM), jnp.float32),
        scratch_types=[
            pltpu.VMEM((nseg, _CH), jnp.int32),
            pltpu.VMEM((b_per_w, _DIM), jnp.float32),
            pltpu.SemaphoreType.DMA,
        ],
    )
    def k(table_hbm, idx_hbm, out_hbm, idx_v, rows_v, sem):
        wid = lax.axis_index("s") * info.num_cores + lax.axis_index("c")
        pltpu.sync_copy(idx_hbm.at[wid], idx_v)
        copies = [
            pltpu.async_copy(
                table_hbm.at[idx_v.at[s]],
                rows_v.at[pl.ds(s * _CH, _CH)],
                sem,
            )
            for s in range(nseg)
        ]
        for c in copies:
            c.wait()
        pltpu.sync_copy(rows_v, out_hbm.at[pl.ds(wid * b_per_w, b_per_w)])

    return k(table, idx3)


def kernel(inputs, codebook):
    original_shape = inputs.shape
    x = inputs.reshape(-1, _DIM)
    idx, cbt = _tc_argmin(x, codebook)
    q = _sc_gather(cbt, idx)
    return q.reshape(original_shape)


# speedup vs baseline: 1.0241x; 1.0080x over previous
import functools

import jax
import jax.numpy as jnp
from jax import lax
from jax.experimental import pallas as pl
from jax.experimental.pallas import tpu as pltpu
from jax.experimental.pallas import tpu_sc as plsc

_NUM_CODES = 8192
_DIM = 32
_TB = 256
_CH = 128


def _vq_argmin_block(x_ref, cb_ref, idx_ref, cbt_ref, b_ref, cbm2_ref):
    @pl.when(pl.program_id(0) == 0)
    def _():
        cb0 = cb_ref[...]
        b_ref[...] = jnp.sum(cb0 * cb0, axis=0, keepdims=True)
        cbm2_ref[...] = -2.0 * cb0
        cbt_ref[...] = cb0.T

    x = x_ref[...]
    a = jnp.sum(x * x, axis=1, keepdims=True)
    m2 = jnp.dot(x, cbm2_ref[...], preferred_element_type=jnp.float32)
    b = b_ref[...]

    nchunks = _NUM_CODES // _CH
    run_min = jnp.full((_TB, _CH), jnp.inf, dtype=jnp.float32)
    run_chunk = jnp.zeros((_TB, _CH), dtype=jnp.int32)
    for j in range(nchunks):
        sl = slice(j * _CH, (j + 1) * _CH)
        nrm = (a + b[:, sl]) + m2[:, sl]
        lt = nrm < run_min
        run_min = jnp.where(lt, nrm, run_min)
        run_chunk = jnp.where(lt, j, run_chunk)

    gmin = jnp.min(run_min, axis=1, keepdims=True)
    lane = jax.lax.broadcasted_iota(jnp.int32, (_TB, _CH), 1)
    cand = run_chunk * _CH + lane
    idx = jnp.min(jnp.where(run_min == gmin, cand, _NUM_CODES), axis=1)
    idx_ref[...] = idx.reshape(1, 1, _TB)


def _tc_argmin(x, codebook):
    n = x.shape[0]
    grid = (n // _TB,)
    idx, cbt = pl.pallas_call(
        _vq_argmin_block,
        grid=grid,
        in_specs=[
            pl.BlockSpec((_TB, _DIM), lambda i: (i, 0)),
            pl.BlockSpec((_DIM, _NUM_CODES), lambda i: (0, 0)),
        ],
        out_specs=[
            pl.BlockSpec((1, 1, _TB), lambda i: (i, 0, 0)),
            pl.BlockSpec((_NUM_CODES, _DIM), lambda i: (0, 0)),
        ],
        out_shape=(
            jax.ShapeDtypeStruct((n // _TB, 1, _TB), jnp.int32),
            jax.ShapeDtypeStruct((_NUM_CODES, _DIM), jnp.float32),
        ),
        scratch_shapes=[
            pltpu.VMEM((1, _NUM_CODES), jnp.float32),
            pltpu.VMEM((_DIM, _NUM_CODES), jnp.float32),
        ],
    )(x, codebook)
    return idx.reshape(n), cbt


def _sc_gather(table, idx):
    B = idx.shape[0]
    info = plsc.get_sparse_core_info()
    nw = info.num_cores * info.num_subcores
    b_per_w = B // nw
    nseg = b_per_w // _CH
    idx3 = idx.reshape(nw, nseg, _CH)
    mesh = plsc.VectorSubcoreMesh(core_axis_name="c", subcore_axis_name="s")

    @functools.partial(
        pl.kernel, mesh=mesh,
        compiler_params=pltpu.CompilerParams(use_tc_tiling_on_sc=False),
        out_type=jax.ShapeDtypeStruct((B, _DIM), jnp.float32),
        scratch_types=[
            pltpu.VMEM((nseg, _CH), jnp.int32),
            pltpu.VMEM((b_per_w, _DIM), jnp.float32),
            pltpu.SemaphoreType.DMA,
        ],
    )
    def k(table_hbm, idx_hbm, out_hbm, idx_v, rows_v, sem):
        wid = lax.axis_index("s") * info.num_cores + lax.axis_index("c")
        pltpu.sync_copy(idx_hbm.at[wid], idx_v)
        copies = [
            pltpu.async_copy(
                table_hbm.at[idx_v.at[s]],
                rows_v.at[pl.ds(s * _CH, _CH)],
                sem,
            )
            for s in range(nseg)
        ]
        for c in copies:
            c.wait()
        pltpu.sync_copy(rows_v, out_hbm.at[pl.ds(wid * b_per_w, b_per_w)])

    return k(table, idx3)


def kernel(inputs, codebook):
    original_shape = inputs.shape
    x = inputs.reshape(-1, _DIM)
    idx, cbt = _tc_argmin(x, codebook)
    q = _sc_gather(cbt, idx)
    return q.reshape(original_shape)


# speedup vs baseline: 1.0968x; 1.0710x over previous
import functools

import jax
import jax.numpy as jnp
from jax import lax
from jax.experimental import pallas as pl
from jax.experimental.pallas import tpu as pltpu
from jax.experimental.pallas import tpu_sc as plsc

_NUM_CODES = 8192
_DIM = 32
_TB = 256
_CH = 128


def _vq_argmin_block(x_ref, cb_ref, idx_ref, b_ref, cbm2_ref):
    @pl.when(pl.program_id(0) == 0)
    def _():
        cb0 = cb_ref[...]
        b_ref[...] = jnp.sum(cb0 * cb0, axis=0, keepdims=True)
        cbm2_ref[...] = -2.0 * cb0

    x = x_ref[...]
    a = jnp.sum(x * x, axis=1, keepdims=True)
    m2 = jnp.dot(x, cbm2_ref[...], preferred_element_type=jnp.float32)
    b = b_ref[...]

    nchunks = _NUM_CODES // _CH
    run_min = jnp.full((_TB, _CH), jnp.inf, dtype=jnp.float32)
    run_chunk = jnp.zeros((_TB, _CH), dtype=jnp.int32)
    for j in range(nchunks):
        sl = slice(j * _CH, (j + 1) * _CH)
        nrm = (a + b[:, sl]) + m2[:, sl]
        lt = nrm < run_min
        run_min = jnp.where(lt, nrm, run_min)
        run_chunk = jnp.where(lt, j, run_chunk)

    gmin = jnp.min(run_min, axis=1, keepdims=True)
    lane = jax.lax.broadcasted_iota(jnp.int32, (_TB, _CH), 1)
    cand = run_chunk * _CH + lane
    idx = jnp.min(jnp.where(run_min == gmin, cand, _NUM_CODES), axis=1)
    idx_ref[...] = idx.reshape(1, 1, _TB)


def _tc_argmin(x, codebook):
    n = x.shape[0]
    grid = (n // _TB,)
    idx = pl.pallas_call(
        _vq_argmin_block,
        grid=grid,
        in_specs=[
            pl.BlockSpec((_TB, _DIM), lambda i: (i, 0)),
            pl.BlockSpec((_DIM, _NUM_CODES), lambda i: (0, 0)),
        ],
        out_specs=pl.BlockSpec((1, 1, _TB), lambda i: (i, 0, 0)),
        out_shape=jax.ShapeDtypeStruct((n // _TB, 1, _TB), jnp.int32),
        scratch_shapes=[
            pltpu.VMEM((1, _NUM_CODES), jnp.float32),
            pltpu.VMEM((_DIM, _NUM_CODES), jnp.float32),
        ],
    )(x, codebook)
    return idx.reshape(n)


def _sc_gather(table, idx):
    B = idx.shape[0]
    info = plsc.get_sparse_core_info()
    nw = info.num_cores * info.num_subcores
    b_per_w = B // nw
    nseg = b_per_w // _CH
    idx3 = idx.reshape(nw, nseg, _CH)
    mesh = plsc.VectorSubcoreMesh(core_axis_name="c", subcore_axis_name="s")

    @functools.partial(
        pl.kernel, mesh=mesh,
        compiler_params=pltpu.CompilerParams(use_tc_tiling_on_sc=False),
        out_type=jax.ShapeDtypeStruct((B, _DIM), jnp.float32),
        scratch_types=[
            pltpu.VMEM((nseg, _CH), jnp.int32),
            pltpu.VMEM((b_per_w, _DIM), jnp.float32),
            pltpu.SemaphoreType.DMA,
        ],
    )
    def k(table_hbm, idx_hbm, out_hbm, idx_v, rows_v, sem):
        wid = lax.axis_index("s") * info.num_cores + lax.axis_index("c")
        pltpu.sync_copy(idx_hbm.at[wid], idx_v)
        copies = [
            pltpu.async_copy(
                table_hbm.at[idx_v.at[s]],
                rows_v.at[pl.ds(s * _CH, _CH)],
                sem,
            )
            for s in range(nseg)
        ]
        for c in copies:
            c.wait()
        pltpu.sync_copy(rows_v, out_hbm.at[pl.ds(wid * b_per_w, b_per_w)])

    return k(table, idx3)


def _tc_gather_body(idx_ref, cbt_ref, out_ref):
    def outer(t8, carry):
        base = t8 * 8
        for u in range(8):
            i = idx_ref[base + u]
            out_ref[pl.ds(base + u, 1), :] = cbt_ref[pl.ds(i, 1), :]
        return carry
    lax.fori_loop(0, idx_ref.shape[0] // 8, outer, 0)


def _tc_gather(cbt, idx):
    n = idx.shape[0]
    return pl.pallas_call(
        _tc_gather_body,
        in_specs=[
            pl.BlockSpec(memory_space=pltpu.SMEM),
            pl.BlockSpec(memory_space=pltpu.ANY if False else pltpu.VMEM),
        ],
        out_specs=pl.BlockSpec(memory_space=pltpu.VMEM),
        out_shape=jax.ShapeDtypeStruct((n, _DIM), jnp.float32),
    )(idx, cbt)


def kernel(inputs, codebook):
    original_shape = inputs.shape
    x = inputs.reshape(-1, _DIM)
    idx = _tc_argmin(x, codebook)
    q = _tc_gather(codebook.T, idx)
    return q.reshape(original_shape)


# speedup vs baseline: 1.1196x; 1.0208x over previous
import jax
import jax.numpy as jnp
from jax import lax
from jax.experimental import pallas as pl
from jax.experimental.pallas import tpu as pltpu

_NUM_CODES = 8192
_DIM = 32
_TB = 256
_CH = 128
_LAG = 8


def _vq_body(x_ref, cb_ref, idx_smem, idx_ref, q_ref, b_ref, cbm2_ref, cbt_ref):
    s = pl.program_id(0)
    nblocks = pl.num_programs(0) - _LAG

    @pl.when(s == 0)
    def _():
        cb0 = cb_ref[...]
        b_ref[...] = jnp.sum(cb0 * cb0, axis=0, keepdims=True)
        cbm2_ref[...] = -2.0 * cb0
        cbt_ref[...] = cb0.T

    @pl.when(s < nblocks)
    def _():
        x = x_ref[...]
        a = jnp.sum(x * x, axis=1, keepdims=True)
        m2 = jnp.dot(x, cbm2_ref[...], preferred_element_type=jnp.float32)
        b = b_ref[...]

        nchunks = _NUM_CODES // _CH
        run_min = jnp.full((_TB, _CH), jnp.inf, dtype=jnp.float32)
        run_chunk = jnp.zeros((_TB, _CH), dtype=jnp.int32)
        for j in range(nchunks):
            sl = slice(j * _CH, (j + 1) * _CH)
            nrm = (a + b[:, sl]) + m2[:, sl]
            lt = nrm < run_min
            run_min = jnp.where(lt, nrm, run_min)
            run_chunk = jnp.where(lt, j, run_chunk)

        gmin = jnp.min(run_min, axis=1, keepdims=True)
        lane = jax.lax.broadcasted_iota(jnp.int32, (_TB, _CH), 1)
        cand = run_chunk * _CH + lane
        idx = jnp.min(jnp.where(run_min == gmin, cand, _NUM_CODES), axis=1)
        idx_ref[...] = idx.reshape(1, 1, _TB)

    @pl.when(s >= _LAG)
    def _():
        def outer(t8, carry):
            base = t8 * 8
            for u in range(8):
                i = idx_smem[0, 0, base + u]
                q_ref[pl.ds(base + u, 1), :] = cbt_ref[pl.ds(i, 1), :]
            return carry
        lax.fori_loop(0, _TB // 8, outer, 0)


def kernel(inputs, codebook):
    original_shape = inputs.shape
    x = inputs.reshape(-1, _DIM)
    n = x.shape[0]
    nblocks = n // _TB
    nsteps = nblocks + _LAG
    idx_seed = jnp.zeros((nblocks + 1, 1, _TB), jnp.int32)
    _, q = pl.pallas_call(
        _vq_body,
        grid=(nsteps,),
        in_specs=[
            pl.BlockSpec((_TB, _DIM), lambda s: (jnp.minimum(s, nblocks - 1), 0)),
            pl.BlockSpec((_DIM, _NUM_CODES), lambda s: (0, 0)),
            pl.BlockSpec((1, 1, _TB),
                         lambda s: (jnp.where(s >= _LAG, s - _LAG, nblocks), 0, 0),
                         memory_space=pltpu.SMEM),
        ],
        out_specs=[
            pl.BlockSpec((1, 1, _TB), lambda s: (jnp.minimum(s, nblocks), 0, 0)),
            pl.BlockSpec((_TB, _DIM), lambda s: (jnp.maximum(s - _LAG, 0), 0)),
        ],
        out_shape=(
            jax.ShapeDtypeStruct((nblocks + 1, 1, _TB), jnp.int32),
            jax.ShapeDtypeStruct((n, _DIM), jnp.float32),
        ),
        scratch_shapes=[
            pltpu.VMEM((1, _NUM_CODES), jnp.float32),
            pltpu.VMEM((_DIM, _NUM_CODES), jnp.float32),
            pltpu.VMEM((_NUM_CODES, _DIM), jnp.float32),
        ],
        input_output_aliases={2: 0},
    )(x, codebook, idx_seed)
    return q.reshape(original_shape)


# speedup vs baseline: 1.1642x; 1.0398x over previous
import jax
import jax.numpy as jnp
from jax import lax
from jax.experimental import pallas as pl
from jax.experimental.pallas import tpu as pltpu

_NUM_CODES = 8192
_DIM = 32
_TB = 256
_CH = 128
_LAG = 8


def _vq_body(x_ref, cb_ref, idx_smem, idx_ref, q_ref, b_ref, cbm2_ref, cbt_ref):
    s = pl.program_id(0)
    nblocks = pl.num_programs(0) - _LAG

    @pl.when(s == 0)
    def _():
        cb0 = cb_ref[...]
        b_ref[...] = jnp.sum(cb0 * cb0, axis=0, keepdims=True)
        cbm2_ref[...] = -2.0 * cb0
        cbt_ref[...] = cb0.T

    def _gather_tokens(base, count):
        for u in range(count):
            i = idx_smem[0, 0, base + u]
            q_ref[pl.ds(base + u, 1), :] = cbt_ref[pl.ds(i, 1), :]

    @pl.when(s < nblocks)
    def _():
        x = x_ref[...]
        a = jnp.sum(x * x, axis=1, keepdims=True)
        m2 = jnp.dot(x, cbm2_ref[...], preferred_element_type=jnp.float32)
        b = b_ref[...]

        nchunks = _NUM_CODES // _CH
        per_chunk = _TB // nchunks
        run_min = jnp.full((_TB, _CH), jnp.inf, dtype=jnp.float32)
        run_chunk = jnp.zeros((_TB, _CH), dtype=jnp.int32)
        for j in range(nchunks):
            sl = slice(j * _CH, (j + 1) * _CH)
            nrm = (a + b[:, sl]) + m2[:, sl]
            lt = nrm < run_min
            run_min = jnp.where(lt, nrm, run_min)
            run_chunk = jnp.where(lt, j, run_chunk)
            _gather_tokens(j * per_chunk, per_chunk)

        gmin = jnp.min(run_min, axis=1, keepdims=True)
        lane = jax.lax.broadcasted_iota(jnp.int32, (_TB, _CH), 1)
        cand = run_chunk * _CH + lane
        idx = jnp.min(jnp.where(run_min == gmin, cand, _NUM_CODES), axis=1)
        idx_ref[...] = idx.reshape(1, 1, _TB)

    @pl.when(s >= nblocks)
    def _():
        def outer(t8, carry):
            _gather_tokens(t8 * 8, 8)
            return carry
        lax.fori_loop(0, _TB // 8, outer, 0)


def kernel(inputs, codebook):
    original_shape = inputs.shape
    x = inputs.reshape(-1, _DIM)
    n = x.shape[0]
    nblocks = n // _TB
    nsteps = nblocks + _LAG
    idx_seed = jnp.zeros((nblocks + 1, 1, _TB), jnp.int32)
    _, q = pl.pallas_call(
        _vq_body,
        grid=(nsteps,),
        in_specs=[
            pl.BlockSpec((_TB, _DIM), lambda s: (jnp.minimum(s, nblocks - 1), 0)),
            pl.BlockSpec((_DIM, _NUM_CODES), lambda s: (0, 0)),
            pl.BlockSpec((1, 1, _TB),
                         lambda s: (jnp.where(s >= _LAG, s - _LAG, nblocks), 0, 0),
                         memory_space=pltpu.SMEM),
        ],
        out_specs=[
            pl.BlockSpec((1, 1, _TB), lambda s: (jnp.minimum(s, nblocks), 0, 0)),
            pl.BlockSpec((_TB, _DIM), lambda s: (jnp.maximum(s - _LAG, 0), 0)),
        ],
        out_shape=(
            jax.ShapeDtypeStruct((nblocks + 1, 1, _TB), jnp.int32),
            jax.ShapeDtypeStruct((n, _DIM), jnp.float32),
        ),
        scratch_shapes=[
            pltpu.VMEM((1, _NUM_CODES), jnp.float32),
            pltpu.VMEM((_DIM, _NUM_CODES), jnp.float32),
            pltpu.VMEM((_NUM_CODES, _DIM), jnp.float32),
        ],
        input_output_aliases={2: 0},
    )(x, codebook, idx_seed)
    return q.reshape(original_shape)


# speedup vs baseline: 1.1912x; 1.0232x over previous
import jax
import jax.numpy as jnp
from jax import lax
from jax.experimental import pallas as pl
from jax.experimental.pallas import tpu as pltpu

_NUM_CODES = 8192
_DIM = 32
_TB = 256
_CH = 128
_LAG = 4


def _vq_body(x_ref, cb_ref, idx_smem, idx_ref, q_ref, b_ref, cbm2_ref, cbt_ref):
    s = pl.program_id(0)
    nblocks = pl.num_programs(0) - _LAG

    @pl.when(s == 0)
    def _():
        cb0 = cb_ref[...]
        b_ref[...] = jnp.sum(cb0 * cb0, axis=0, keepdims=True)
        cbm2_ref[...] = -2.0 * cb0
        cbt_ref[...] = cb0.T

    def _gather_tokens(base, count):
        for u in range(count):
            i = idx_smem[0, 0, base + u]
            q_ref[pl.ds(base + u, 1), :] = cbt_ref[pl.ds(i, 1), :]

    @pl.when(s < nblocks)
    def _():
        x = x_ref[...]
        a = jnp.sum(x * x, axis=1, keepdims=True)
        m2 = jnp.dot(x, cbm2_ref[...], preferred_element_type=jnp.float32)
        b = b_ref[...]

        nchunks = _NUM_CODES // _CH
        per_chunk = _TB // nchunks
        run_min = jnp.full((_TB, _CH), jnp.inf, dtype=jnp.float32)
        run_chunk = jnp.zeros((_TB, _CH), dtype=jnp.int32)
        for j in range(nchunks):
            sl = slice(j * _CH, (j + 1) * _CH)
            nrm = (a + b[:, sl]) + m2[:, sl]
            lt = nrm < run_min
            run_min = jnp.where(lt, nrm, run_min)
            run_chunk = jnp.where(lt, j, run_chunk)
            _gather_tokens(j * per_chunk, per_chunk)

        gmin = jnp.min(run_min, axis=1, keepdims=True)
        lane = jax.lax.broadcasted_iota(jnp.int32, (_TB, _CH), 1)
        cand = run_chunk * _CH + lane
        idx = jnp.min(jnp.where(run_min == gmin, cand, _NUM_CODES), axis=1)
        idx_ref[...] = idx.reshape(1, 1, _TB)

    @pl.when(s >= nblocks)
    def _():
        def outer(t8, carry):
            _gather_tokens(t8 * 8, 8)
            return carry
        lax.fori_loop(0, _TB // 8, outer, 0)


def kernel(inputs, codebook):
    original_shape = inputs.shape
    x = inputs.reshape(-1, _DIM)
    n = x.shape[0]
    nblocks = n // _TB
    nsteps = nblocks + _LAG
    idx_seed = jnp.zeros((nblocks + 1, 1, _TB), jnp.int32)
    _, q = pl.pallas_call(
        _vq_body,
        grid=(nsteps,),
        in_specs=[
            pl.BlockSpec((_TB, _DIM), lambda s: (jnp.minimum(s, nblocks - 1), 0)),
            pl.BlockSpec((_DIM, _NUM_CODES), lambda s: (0, 0)),
            pl.BlockSpec((1, 1, _TB),
                         lambda s: (jnp.where(s >= _LAG, s - _LAG, nblocks), 0, 0),
                         memory_space=pltpu.SMEM),
        ],
        out_specs=[
            pl.BlockSpec((1, 1, _TB), lambda s: (jnp.minimum(s, nblocks), 0, 0)),
            pl.BlockSpec((_TB, _DIM), lambda s: (jnp.maximum(s - _LAG, 0), 0)),
        ],
        out_shape=(
            jax.ShapeDtypeStruct((nblocks + 1, 1, _TB), jnp.int32),
            jax.ShapeDtypeStruct((n, _DIM), jnp.float32),
        ),
        scratch_shapes=[
            pltpu.VMEM((1, _NUM_CODES), jnp.float32),
            pltpu.VMEM((_DIM, _NUM_CODES), jnp.float32),
            pltpu.VMEM((_NUM_CODES, _DIM), jnp.float32),
        ],
        input_output_aliases={2: 0},
    )(x, codebook, idx_seed)
    return q.reshape(original_shape)
